# baseline (speedup 1.0000x reference)
import functools

import jax
import jax.numpy as jnp
from jax import lax
from jax.experimental import pallas as pl
from jax.experimental.pallas import tpu as pltpu
from jax.experimental.pallas import tpu_sc as plsc

N = 10000
E = 320000
D = 128

NC = 2
NS = 16
NW = NC * NS
EPW = E // NW
CHUNK = 80
NCHUNK = EPW // CHUNK


def _proj_body(x_ref, wst_ref, wdt_ref, ps_ref, pd_ref):
    xv = x_ref[...]
    ps_ref[...] = jnp.dot(xv, wst_ref[...], preferred_element_type=jnp.float32)
    pd_ref[...] = jnp.dot(xv, wdt_ref[...], preferred_element_type=jnp.float32)


def _node_projections(x, wst, wdt):
    return pl.pallas_call(
        _proj_body,
        out_shape=(
            jax.ShapeDtypeStruct((N, D), jnp.float32),
            jax.ShapeDtypeStruct((N, D), jnp.float32),
        ),
    )(x, wst, wdt)


def _sc_body(ps_hbm, pd_hbm, src_hbm, dst_hbm, out_hbm,
             src_v, dst_v, bs0, bd0, bs1, bd1, ss0, sd0, ss1, sd1, so0, so1):
    wid = lax.axis_index("s") * NC + lax.axis_index("c")
    base = wid * EPW
    pltpu.sync_copy(src_hbm.at[pl.ds(base, EPW)], src_v)
    pltpu.sync_copy(dst_hbm.at[pl.ds(base, EPW)], dst_v)

    def start(ci, bs, bd, ss, sd):
        off = pl.multiple_of(ci * CHUNK, CHUNK)
        pltpu.async_copy(ps_hbm.at[src_v.at[pl.ds(off, CHUNK)]], bs, ss)
        pltpu.async_copy(pd_hbm.at[dst_v.at[pl.ds(off, CHUNK)]], bd, sd)

    def add_store(ci, bs, bd, ss, sd, so):
        off = pl.multiple_of(ci * CHUNK, CHUNK)
        pltpu.make_async_copy(ps_hbm.at[src_v.at[pl.ds(off, CHUNK)]], bs, ss).wait()
        pltpu.make_async_copy(pd_hbm.at[dst_v.at[pl.ds(off, CHUNK)]], bd, sd).wait()

        def add_row(r, _):
            for j in range(D // 16):
                sl = pl.ds(j * 16, 16)
                plsc.addupdate(bs.at[r, sl], bd[r, sl])
            return 0

        lax.fori_loop(0, CHUNK, add_row, 0, unroll=4)
        pltpu.async_copy(bs, out_hbm.at[pl.ds(base + off, CHUNK)], so)

    def wait_store(ci, bs, so):
        off = pl.multiple_of(ci * CHUNK, CHUNK)
        pltpu.make_async_copy(bs, out_hbm.at[pl.ds(base + off, CHUNK)], so).wait()

    start(0, bs0, bd0, ss0, sd0)

    def pair_body(g, _):
        ci0 = g * 2

        @pl.when(g > 0)
        def _():
            wait_store(ci0 - 1, bs1, so1)

        start(ci0 + 1, bs1, bd1, ss1, sd1)
        add_store(ci0, bs0, bd0, ss0, sd0, so0)
        add_store(ci0 + 1, bs1, bd1, ss1, sd1, so1)
        wait_store(ci0, bs0, so0)
        start(ci0 + 2, bs0, bd0, ss0, sd0)
        return 0

    npair = (NCHUNK - 1) // 2
    lax.fori_loop(0, npair, pair_body, 0)
    wait_store(NCHUNK - 2, bs1, so1)
    add_store(NCHUNK - 1, bs0, bd0, ss0, sd0, so0)
    wait_store(NCHUNK - 1, bs0, so0)


def _edge_gather_add(ps, pd, src, dst):
    mesh = plsc.VectorSubcoreMesh(core_axis_name="c", subcore_axis_name="s")
    return pl.kernel(
        _sc_body,
        out_type=jax.ShapeDtypeStruct((E, D), jnp.float32),
        mesh=mesh,
        scratch_types=[
            pltpu.VMEM((EPW,), jnp.int32),
            pltpu.VMEM((EPW,), jnp.int32),
            pltpu.VMEM((CHUNK, D), jnp.float32),
            pltpu.VMEM((CHUNK, D), jnp.float32),
            pltpu.VMEM((CHUNK, D), jnp.float32),
            pltpu.VMEM((CHUNK, D), jnp.float32),
            pltpu.SemaphoreType.DMA,
            pltpu.SemaphoreType.DMA,
            pltpu.SemaphoreType.DMA,
            pltpu.SemaphoreType.DMA,
            pltpu.SemaphoreType.DMA,
            pltpu.SemaphoreType.DMA,
        ],
    )(ps, pd, src, dst)


BK = 3200


def _final_body(e_ref, wet_ref, b_ref, g_ref, out_ref):
    out_ref[...] = (
        jnp.dot(e_ref[...], wet_ref[...], preferred_element_type=jnp.float32)
        + g_ref[...]
        + b_ref[...]
    )


def _final(e, wet, b2, g):
    grid = (E // BK,)
    return pl.pallas_call(
        _final_body,
        grid=grid,
        in_specs=[
            pl.BlockSpec((BK, 16), lambda i: (i, 0)),
            pl.BlockSpec((16, D), lambda i: (0, 0)),
            pl.BlockSpec((1, D), lambda i: (0, 0)),
            pl.BlockSpec((BK, D), lambda i: (i, 0)),
        ],
        out_specs=pl.BlockSpec((BK, D), lambda i: (i, 0)),
        out_shape=jax.ShapeDtypeStruct((E, D), jnp.float32),
    )(e, wet, b2, g)


def kernel(x, e, edge_index, W, b):
    wet = W[:, :16].T
    wst = W[:, 16:16 + D].T
    wdt = W[:, 16 + D:].T
    src = edge_index[0]
    dst = edge_index[1]
    ps, pd = _node_projections(x, wst, wdt)
    g = _edge_gather_add(ps, pd, src, dst)
    return _final(e, wet, b.reshape(1, D), g)


# speedup vs baseline: 1.0364x; 1.0364x over previous
import functools

import jax
import jax.numpy as jnp
from jax import lax
from jax.experimental import pallas as pl
from jax.experimental.pallas import tpu as pltpu
from jax.experimental.pallas import tpu_sc as plsc

N = 10000
E = 320000
D = 128

NC = 2
NS = 16
NW = NC * NS
EPW = E // NW
CHUNK = 80
NCHUNK = EPW // CHUNK


def _proj_body(x_ref, wst_ref, wdt_ref, ps_ref, pd_ref):
    xv = x_ref[...]
    ps_ref[...] = jnp.dot(xv, wst_ref[...], preferred_element_type=jnp.float32)
    pd_ref[...] = jnp.dot(xv, wdt_ref[...], preferred_element_type=jnp.float32)


def _node_projections(x, wst, wdt):
    return pl.pallas_call(
        _proj_body,
        out_shape=(
            jax.ShapeDtypeStruct((N, D), jnp.float32),
            jax.ShapeDtypeStruct((N, D), jnp.float32),
        ),
    )(x, wst, wdt)


def _sc_body(ps_hbm, pd_hbm, src_hbm, dst_hbm, out_hbm,
             src_v, dst_v, bs0, bd0, bs1, bd1, ss0, sd0, ss1, sd1):
    wid = lax.axis_index("s") * NC + lax.axis_index("c")
    base = wid * EPW
    pltpu.sync_copy(src_hbm.at[pl.ds(base, EPW)], src_v)
    pltpu.sync_copy(dst_hbm.at[pl.ds(base, EPW)], dst_v)

    def start(ci, bs, bd, ss, sd):
        off = pl.multiple_of(ci * CHUNK, CHUNK)
        pltpu.async_copy(ps_hbm.at[src_v.at[pl.ds(off, CHUNK)]], bs, ss)
        pltpu.async_copy(pd_hbm.at[dst_v.at[pl.ds(off, CHUNK)]], bd, sd)

    def finish(ci, bs, bd, ss, sd):
        off = pl.multiple_of(ci * CHUNK, CHUNK)
        pltpu.make_async_copy(ps_hbm.at[src_v.at[pl.ds(off, CHUNK)]], bs, ss).wait()
        pltpu.make_async_copy(pd_hbm.at[dst_v.at[pl.ds(off, CHUNK)]], bd, sd).wait()

        @plsc.parallel_loop(0, CHUNK, 1, unroll=8)
        def add_row(r):
            for j in range(D // 16):
                sl = pl.ds(j * 16, 16)
                plsc.addupdate(bs.at[r, sl], bd[r, sl])
        pltpu.sync_copy(bs, out_hbm.at[pl.ds(base + off, CHUNK)])

    start(0, bs0, bd0, ss0, sd0)

    def pair_body(g, _):
        ci0 = g * 2
        start(ci0 + 1, bs1, bd1, ss1, sd1)
        finish(ci0, bs0, bd0, ss0, sd0)
        start(ci0 + 2, bs0, bd0, ss0, sd0)
        finish(ci0 + 1, bs1, bd1, ss1, sd1)
        return 0

    lax.fori_loop(0, (NCHUNK - 1) // 2, pair_body, 0)
    finish(NCHUNK - 1, bs0, bd0, ss0, sd0)


def _edge_gather_add(ps, pd, src, dst):
    mesh = plsc.VectorSubcoreMesh(core_axis_name="c", subcore_axis_name="s")
    return pl.kernel(
        _sc_body,
        out_type=jax.ShapeDtypeStruct((E, D), jnp.float32),
        mesh=mesh,
        scratch_types=[
            pltpu.VMEM((EPW,), jnp.int32),
            pltpu.VMEM((EPW,), jnp.int32),
            pltpu.VMEM((CHUNK, D), jnp.float32),
            pltpu.VMEM((CHUNK, D), jnp.float32),
            pltpu.VMEM((CHUNK, D), jnp.float32),
            pltpu.VMEM((CHUNK, D), jnp.float32),
            pltpu.SemaphoreType.DMA,
            pltpu.SemaphoreType.DMA,
            pltpu.SemaphoreType.DMA,
            pltpu.SemaphoreType.DMA,
        ],
    )(ps, pd, src, dst)


BK = 3200


def _final_body(e_ref, wet_ref, b_ref, g_ref, out_ref):
    out_ref[...] = (
        jnp.dot(e_ref[...], wet_ref[...], preferred_element_type=jnp.float32)
        + g_ref[...]
        + b_ref[...]
    )


def _final(e, wet, b2, g):
    grid = (E // BK,)
    return pl.pallas_call(
        _final_body,
        grid=grid,
        in_specs=[
            pl.BlockSpec((BK, 16), lambda i: (i, 0)),
            pl.BlockSpec((16, D), lambda i: (0, 0)),
            pl.BlockSpec((1, D), lambda i: (0, 0)),
            pl.BlockSpec((BK, D), lambda i: (i, 0)),
        ],
        out_specs=pl.BlockSpec((BK, D), lambda i: (i, 0)),
        out_shape=jax.ShapeDtypeStruct((E, D), jnp.float32),
    )(e, wet, b2, g)


def kernel(x, e, edge_index, W, b):
    wet = W[:, :16].T
    wst = W[:, 16:16 + D].T
    wdt = W[:, 16 + D:].T
    src = edge_index[0]
    dst = edge_index[1]
    ps, pd = _node_projections(x, wst, wdt)
    g = _edge_gather_add(ps, pd, src, dst)
    return _final(e, wet, b.reshape(1, D), g)


# speedup vs baseline: 1.0462x; 1.0095x over previous
import functools

import jax
import jax.numpy as jnp
from jax import lax
from jax.experimental import pallas as pl
from jax.experimental.pallas import tpu as pltpu
from jax.experimental.pallas import tpu_sc as plsc

N = 10000
E = 320000
D = 128

NC = 2
NS = 16
NW = NC * NS
EPW = E // NW
CHUNK = 80
NCHUNK = EPW // CHUNK


def _proj_body(x_ref, wst_ref, wdt_ref, ps_ref, pd_ref):
    xv = x_ref[...]
    ps_ref[...] = jnp.dot(xv, wst_ref[...], preferred_element_type=jnp.float32)
    pd_ref[...] = jnp.dot(xv, wdt_ref[...], preferred_element_type=jnp.float32)


def _node_projections(x, wst, wdt):
    return pl.pallas_call(
        _proj_body,
        out_shape=(
            jax.ShapeDtypeStruct((N, D), jnp.float32),
            jax.ShapeDtypeStruct((N, D), jnp.float32),
        ),
    )(x, wst, wdt)


def _sc_body(ps_hbm, pd_hbm, src_hbm, dst_hbm, out_hbm, src_v, dst_v,
             bs0, bd0, bs1, bd1, bs2, bd2,
             ss0, sd0, ss1, sd1, ss2, sd2, so0, so1, so2):
    wid = lax.axis_index("s") * NC + lax.axis_index("c")
    base = wid * EPW
    pltpu.sync_copy(src_hbm.at[pl.ds(base, EPW)], src_v)
    pltpu.sync_copy(dst_hbm.at[pl.ds(base, EPW)], dst_v)

    slots = ((bs0, bd0, ss0, sd0, so0),
             (bs1, bd1, ss1, sd1, so1),
             (bs2, bd2, ss2, sd2, so2))

    def start(ci, sl):
        bs, bd, ss, sd, _ = sl
        off = pl.multiple_of(ci * CHUNK, CHUNK)
        pltpu.async_copy(ps_hbm.at[src_v.at[pl.ds(off, CHUNK)]], bs, ss)
        pltpu.async_copy(pd_hbm.at[dst_v.at[pl.ds(off, CHUNK)]], bd, sd)

    def wait_store(ci, sl):
        bs, _, _, _, so = sl
        off = pl.multiple_of(ci * CHUNK, CHUNK)
        pltpu.make_async_copy(bs, out_hbm.at[pl.ds(base + off, CHUNK)], so).wait()

    def process(ci, sl):
        bs, bd, ss, sd, so = sl
        off = pl.multiple_of(ci * CHUNK, CHUNK)
        pltpu.make_async_copy(ps_hbm.at[src_v.at[pl.ds(off, CHUNK)]], bs, ss).wait()
        pltpu.make_async_copy(pd_hbm.at[dst_v.at[pl.ds(off, CHUNK)]], bd, sd).wait()

        @plsc.parallel_loop(0, CHUNK, 1, unroll=8)
        def add_row(r):
            for j in range(D // 16):
                sl2 = pl.ds(j * 16, 16)
                plsc.addupdate(bs.at[r, sl2], bd[r, sl2])
        pltpu.async_copy(bs, out_hbm.at[pl.ds(base + off, CHUNK)], so)

    start(0, slots[0])
    start(1, slots[1])

    def body(g, _):
        c0 = g * 3
        for k in range(3):
            c = c0 + k
            process(c, slots[k])

            @pl.when(c > 0)
            def _(c=c, k=k):
                wait_store(c - 1, slots[(k + 2) % 3])

            start(c + 2, slots[(k + 2) % 3])
        return 0

    lax.fori_loop(0, (NCHUNK - 2) // 3, body, 0)
    process(NCHUNK - 2, slots[0])
    wait_store(NCHUNK - 3, slots[2])
    process(NCHUNK - 1, slots[1])
    wait_store(NCHUNK - 2, slots[0])
    wait_store(NCHUNK - 1, slots[1])


def _edge_gather_add(ps, pd, src, dst):
    mesh = plsc.VectorSubcoreMesh(core_axis_name="c", subcore_axis_name="s")
    return pl.kernel(
        _sc_body,
        out_type=jax.ShapeDtypeStruct((E, D), jnp.float32),
        mesh=mesh,
        scratch_types=(
            [pltpu.VMEM((EPW,), jnp.int32)] * 2
            + [pltpu.VMEM((CHUNK, D), jnp.float32)] * 6
            + [pltpu.SemaphoreType.DMA] * 9
        ),
    )(ps, pd, src, dst)


BK = 3200


def _final_body(e_ref, wet_ref, b_ref, g_ref, out_ref):
    out_ref[...] = (
        jnp.dot(e_ref[...], wet_ref[...], preferred_element_type=jnp.float32)
        + g_ref[...]
        + b_ref[...]
    )


def _final(e, wet, b2, g):
    grid = (E // BK,)
    return pl.pallas_call(
        _final_body,
        grid=grid,
        in_specs=[
            pl.BlockSpec((BK, 16), lambda i: (i, 0)),
            pl.BlockSpec((16, D), lambda i: (0, 0)),
            pl.BlockSpec((1, D), lambda i: (0, 0)),
            pl.BlockSpec((BK, D), lambda i: (i, 0)),
        ],
        out_specs=pl.BlockSpec((BK, D), lambda i: (i, 0)),
        out_shape=jax.ShapeDtypeStruct((E, D), jnp.float32),
    )(e, wet, b2, g)


def kernel(x, e, edge_index, W, b):
    wet = W[:, :16].T
    wst = W[:, 16:16 + D].T
    wdt = W[:, 16 + D:].T
    src = edge_index[0]
    dst = edge_index[1]
    ps, pd = _node_projections(x, wst, wdt)
    g = _edge_gather_add(ps, pd, src, dst)
    return _final(e, wet, b.reshape(1, D), g)
